# baseline (speedup 1.0000x reference)
import functools

import jax
import jax.numpy as jnp
from jax import lax
from jax.experimental import pallas as pl
from jax.experimental.pallas import tpu as pltpu
from jax.experimental.pallas import tpu_sc as plsc

NC, NS, LANES = 2, 16, 16
NW = NC * NS
KW = 40
SCROWS = 25

N = 10000
E = 320000
D = 128
BN = 1000
NPT = N // NS
RPW = (E // KW) // NW


def _deg_body(dst2, ew2, degp, idx_d, ew_v, zb, acc1, sem_sc):
    c = lax.axis_index("c")
    s = lax.axis_index("s")
    wid = c * NS + s
    @pl.when(s < 5)
    def _():
        @pl.loop(0, zb.shape[0] // LANES)
        def _(i):
            zb[pl.ds(i * LANES, LANES)] = jnp.zeros((LANES,), jnp.float32)

        pltpu.sync_copy(zb, acc1.at[pl.ds(s * 2000, 2000)])

    plsc.subcore_barrier()

    @pl.loop(0, RPW // SCROWS)
    def _(ci):
        blk = wid * (RPW // SCROWS) + ci
        pltpu.sync_copy(dst2.at[blk], idx_d)
        pltpu.sync_copy(ew2.at[blk], ew_v)
        descs = [
            pltpu.async_copy(ew_v.at[j], acc1.at[idx_d.at[j]], sem_sc, add=True)
            for j in range(SCROWS)
        ]
        for dsc in descs:
            dsc.wait()

    plsc.subcore_barrier()

    @pl.when(s == 0)
    def _():
        pltpu.sync_copy(acc1, degp.at[c, 0])


_deg_call = functools.partial(
    pl.kernel,
    out_type=jax.ShapeDtypeStruct((NC, 1, N), jnp.float32),
    mesh=plsc.VectorSubcoreMesh(
        core_axis_name="c", subcore_axis_name="s", num_cores=NC, num_subcores=NS
    ),
    scratch_types=[
        pltpu.VMEM((SCROWS, KW), jnp.int32),
        pltpu.VMEM((SCROWS, KW), jnp.float32),
        pltpu.VMEM((2000,), jnp.float32),
        pltpu.VMEM_SHARED((N,), jnp.float32),
        pltpu.SemaphoreType.DMA,
    ],
    compiler_params=pltpu.CompilerParams(needs_layout_passes=False),
)(_deg_body)


NGB = 4
NSB = 2


def _agg_body(y, src2, dst2, ew1, outp, gb0, gb1, gb2, gb3, sb0, sb1,
              idx_s, idx_d, ew_v, acc,
              sem_g0, sem_g1, sem_g2, sem_g3, sem_s0, sem_s1):
    c = lax.axis_index("c")
    s = lax.axis_index("s")
    wid = c * NS + s

    @pl.loop(0, KW)
    def _(i):
        for r in range(D // LANES):
            gb0[i, pl.ds(r * LANES, LANES)] = jnp.zeros((LANES,), jnp.float32)

    for k in range(NPT // KW):
        pltpu.sync_copy(gb0, acc.at[pl.ds(s * NPT + k * KW, KW)])
    pltpu.sync_copy(
        gb0.at[pl.ds(0, NPT % KW)],
        acc.at[pl.ds(s * NPT + (NPT // KW) * KW, NPT % KW)],
    )

    plsc.subcore_barrier()

    gbufs = (gb0, gb1, gb2, gb3)
    gsems = (sem_g0, sem_g1, sem_g2, sem_g3)
    sbufs = (sb0, sb1)
    ssems = (sem_s0, sem_s1)

    @pl.loop(0, RPW // SCROWS)
    def _(ci):
        blk = wid * (RPW // SCROWS) + ci
        st1 = pltpu.async_copy(src2.at[blk], idx_s, sem_s0)
        st2 = pltpu.async_copy(dst2.at[blk], idx_d, sem_s0)
        st3 = pltpu.async_copy(
            ew1.at[pl.ds(blk * (SCROWS * KW), SCROWS * KW)], ew_v, sem_s0
        )
        st1.wait()
        st2.wait()
        st3.wait()
        gd = [None] * SCROWS
        sd = [None] * SCROWS
        for j in range(NGB - 1):
            gd[j] = pltpu.async_copy(y.at[idx_s.at[j]], gbufs[j % NGB], gsems[j % NGB])
        for j in range(SCROWS):
            gd[j].wait()
            if j >= NSB:
                sd[j - NSB].wait()
            if j + NGB - 1 < SCROWS:
                jn = j + NGB - 1
                gd[jn] = pltpu.async_copy(
                    y.at[idx_s.at[jn]], gbufs[jn % NGB], gsems[jn % NGB]
                )
            gbuf = gbufs[j % NGB]
            sbuf = sbufs[j % NSB]

            @pl.loop(0, KW, unroll=2)
            def _(e):
                ews = plsc.load_gather(
                    ew_v, [jnp.full((LANES,), e, jnp.int32) + (j * KW)]
                )
                for r in range(D // LANES):
                    sbuf[e, pl.ds(r * LANES, LANES)] = (
                        gbuf[e, pl.ds(r * LANES, LANES)] * ews
                    )

            sd[j] = pltpu.async_copy(sbuf, acc.at[idx_d.at[j]], ssems[j % NSB], add=True)
        for j in range(SCROWS - NSB, SCROWS):
            sd[j].wait()

    plsc.subcore_barrier()
    pltpu.sync_copy(acc.at[pl.ds(s * NPT, NPT)], outp.at[c, s])


_agg_call = functools.partial(
    pl.kernel,
    out_type=jax.ShapeDtypeStruct((NC, NS, NPT, D), jnp.float32),
    mesh=plsc.VectorSubcoreMesh(
        core_axis_name="c", subcore_axis_name="s", num_cores=NC, num_subcores=NS
    ),
    scratch_types=[
        pltpu.VMEM((KW, D), jnp.float32),
        pltpu.VMEM((KW, D), jnp.float32),
        pltpu.VMEM((KW, D), jnp.float32),
        pltpu.VMEM((KW, D), jnp.float32),
        pltpu.VMEM((KW, D), jnp.float32),
        pltpu.VMEM((KW, D), jnp.float32),
        pltpu.VMEM((SCROWS, KW), jnp.int32),
        pltpu.VMEM((SCROWS, KW), jnp.int32),
        pltpu.VMEM((SCROWS * KW,), jnp.float32),
        pltpu.VMEM_SHARED((N, D), jnp.float32),
        pltpu.SemaphoreType.DMA,
        pltpu.SemaphoreType.DMA,
        pltpu.SemaphoreType.DMA,
        pltpu.SemaphoreType.DMA,
        pltpu.SemaphoreType.DMA,
        pltpu.SemaphoreType.DMA,
    ],
    compiler_params=pltpu.CompilerParams(needs_layout_passes=False),
)(_agg_body)


def _dense_body(x_ref, w_ref, degp_ref, y_ref, dis_ref):
    dp = degp_ref[...]
    deg = dp[0] + dp[1]
    pos = deg > 0.0
    dis = jnp.where(pos, lax.rsqrt(jnp.where(pos, deg, 1.0)), 0.0)
    xw = jnp.dot(x_ref[...], w_ref[...], preferred_element_type=jnp.float32)
    y_ref[...] = xw * dis
    dis_ref[...] = dis


def _dense_call(x, w, degp3):
    return pl.pallas_call(
        _dense_body,
        grid=(N // BN,),
        in_specs=[
            pl.BlockSpec((BN, D), lambda g: (g, 0)),
            pl.BlockSpec((D, D), lambda g: (0, 0)),
            pl.BlockSpec((NC, BN, 1), lambda g: (0, g, 0)),
        ],
        out_specs=[
            pl.BlockSpec((BN, D), lambda g: (g, 0)),
            pl.BlockSpec((BN, 1), lambda g: (g, 0)),
        ],
        out_shape=[
            jax.ShapeDtypeStruct((N, D), jnp.float32),
            jax.ShapeDtypeStruct((N, 1), jnp.float32),
        ],
    )(x, w, degp3)


def _final_body(outp_ref, dis_ref, b_ref, o_ref):
    t = outp_ref[0] + outp_ref[1]
    o_ref[...] = jnp.maximum(t * dis_ref[...] + b_ref[...], 0.0)


def _final_call(outp, dis, b2):
    return pl.pallas_call(
        _final_body,
        grid=(N // BN,),
        in_specs=[
            pl.BlockSpec((NC, BN, D), lambda g: (0, g, 0)),
            pl.BlockSpec((BN, 1), lambda g: (g, 0)),
            pl.BlockSpec((1, D), lambda g: (0, 0)),
        ],
        out_specs=pl.BlockSpec((BN, D), lambda g: (g, 0)),
        out_shape=jax.ShapeDtypeStruct((N, D), jnp.float32),
    )(outp, dis, b2)


def kernel(x, edge_index, edge_weights, W, b):
    nblk = E // (KW * SCROWS)
    ei = edge_index.astype(jnp.int32)
    src3 = ei[0].reshape(nblk, SCROWS, KW)
    dst3 = ei[1].reshape(nblk, SCROWS, KW)
    ew3 = edge_weights.reshape(nblk, SCROWS, KW)
    degp = _deg_call(dst3, ew3)
    y, dis = _dense_call(x, W, degp.reshape(NC, N, 1))
    outp = _agg_call(y, src3, dst3, edge_weights)
    out = _final_call(outp.reshape(NC, N, D), dis, b.reshape(1, D))
    return (out, edge_index, edge_weights)


# speedup vs baseline: 1.9488x; 1.9488x over previous
import functools

import jax
import jax.numpy as jnp
from jax import lax
from jax.experimental import pallas as pl
from jax.experimental.pallas import tpu as pltpu
from jax.experimental.pallas import tpu_sc as plsc

NC, NS, LANES = 2, 16, 16
NW = NC * NS
KW = 80
SCROWS = 25

N = 10000
E = 320000
D = 128
BN = 1000
NPT = N // NS
RPW = (E // KW) // NW


def _deg_body(dst2, ew2, degp, idx_d, ew_v, zb, acc1, sem_sc):
    c = lax.axis_index("c")
    s = lax.axis_index("s")
    wid = c * NS + s
    @pl.when(s < 5)
    def _():
        @pl.loop(0, zb.shape[0] // LANES)
        def _(i):
            zb[pl.ds(i * LANES, LANES)] = jnp.zeros((LANES,), jnp.float32)

        pltpu.sync_copy(zb, acc1.at[pl.ds(s * 2000, 2000)])

    plsc.subcore_barrier()

    @pl.loop(0, RPW // SCROWS)
    def _(ci):
        blk = wid * (RPW // SCROWS) + ci
        pltpu.sync_copy(dst2.at[blk], idx_d)
        pltpu.sync_copy(ew2.at[blk], ew_v)
        descs = [
            pltpu.async_copy(ew_v.at[j], acc1.at[idx_d.at[j]], sem_sc, add=True)
            for j in range(SCROWS)
        ]
        for dsc in descs:
            dsc.wait()

    plsc.subcore_barrier()

    @pl.when(s == 0)
    def _():
        pltpu.sync_copy(acc1, degp.at[c, 0])


_deg_call = functools.partial(
    pl.kernel,
    out_type=jax.ShapeDtypeStruct((NC, 1, N), jnp.float32),
    mesh=plsc.VectorSubcoreMesh(
        core_axis_name="c", subcore_axis_name="s", num_cores=NC, num_subcores=NS
    ),
    scratch_types=[
        pltpu.VMEM((SCROWS, KW), jnp.int32),
        pltpu.VMEM((SCROWS, KW), jnp.float32),
        pltpu.VMEM((2000,), jnp.float32),
        pltpu.VMEM_SHARED((N,), jnp.float32),
        pltpu.SemaphoreType.DMA,
    ],
    compiler_params=pltpu.CompilerParams(needs_layout_passes=False),
)(_deg_body)


NGB = 3
NSB = 2


def _agg_body(y, src2, dst2, ew1, outp, gb0, gb1,
              idx_s, idx_d, ew_v, acc,
              sem_g0, sem_g1, sem_s0, sem_s1):
    c = lax.axis_index("c")
    s = lax.axis_index("s")
    wid = c * NS + s

    @pl.loop(0, KW)
    def _(i):
        for r in range(D // LANES):
            gb0[i, pl.ds(r * LANES, LANES)] = jnp.zeros((LANES,), jnp.float32)

    for k in range(NPT // KW):
        pltpu.sync_copy(gb0, acc.at[pl.ds(s * NPT + k * KW, KW)])
    pltpu.sync_copy(
        gb0.at[pl.ds(0, NPT % KW)],
        acc.at[pl.ds(s * NPT + (NPT // KW) * KW, NPT % KW)],
    )

    plsc.subcore_barrier()

    gbufs = (gb0, gb1)
    gsems = (sem_g0, sem_g1)
    ssems = (sem_s0, sem_s1)

    @pl.loop(0, RPW // SCROWS)
    def _(ci):
        blk = wid * (RPW // SCROWS) + ci
        st1 = pltpu.async_copy(src2.at[blk], idx_s, sem_s0)
        st2 = pltpu.async_copy(dst2.at[blk], idx_d, sem_s0)
        st3 = pltpu.async_copy(
            ew1.at[pl.ds(blk * (SCROWS * KW), SCROWS * KW)], ew_v, sem_s0
        )
        st1.wait()
        st2.wait()
        st3.wait()
        gd = [None] * SCROWS
        sd = [None] * SCROWS
        gd[0] = pltpu.async_copy(y.at[idx_s.at[0]], gbufs[0], gsems[0])
        for j in range(SCROWS):
            b = j & 1
            gd[j].wait()
            if j >= 1:
                sd[j - 1].wait()
            if j + 1 < SCROWS:
                gd[j + 1] = pltpu.async_copy(
                    y.at[idx_s.at[j + 1]], gbufs[(j + 1) & 1], gsems[(j + 1) & 1]
                )
            gbuf = gbufs[b]

            @pl.loop(0, KW, unroll=2)
            def _(e):
                ews = plsc.load_gather(
                    ew_v, [jnp.full((LANES,), e, jnp.int32) + (j * KW)]
                )
                for r in range(D // LANES):
                    gbuf[e, pl.ds(r * LANES, LANES)] = (
                        gbuf[e, pl.ds(r * LANES, LANES)] * ews
                    )

            sd[j] = pltpu.async_copy(gbuf, acc.at[idx_d.at[j]], ssems[b], add=True)
        sd[SCROWS - 1].wait()

    plsc.subcore_barrier()
    pltpu.sync_copy(acc.at[pl.ds(s * NPT, NPT)], outp.at[c, s])


_agg_call = functools.partial(
    pl.kernel,
    out_type=jax.ShapeDtypeStruct((NC, NS, NPT, D), jnp.float32),
    mesh=plsc.VectorSubcoreMesh(
        core_axis_name="c", subcore_axis_name="s", num_cores=NC, num_subcores=NS
    ),
    scratch_types=[
        pltpu.VMEM((KW, D), jnp.float32),
        pltpu.VMEM((KW, D), jnp.float32),
        pltpu.VMEM((SCROWS, KW), jnp.int32),
        pltpu.VMEM((SCROWS, KW), jnp.int32),
        pltpu.VMEM((SCROWS * KW,), jnp.float32),
        pltpu.VMEM_SHARED((N, D), jnp.float32),
        pltpu.SemaphoreType.DMA,
        pltpu.SemaphoreType.DMA,
        pltpu.SemaphoreType.DMA,
        pltpu.SemaphoreType.DMA,
    ],
    compiler_params=pltpu.CompilerParams(needs_layout_passes=False),
)(_agg_body)


def _dense_body(x_ref, w_ref, degp_ref, y_ref, dis_ref):
    dp = degp_ref[...]
    deg = dp[0] + dp[1]
    pos = deg > 0.0
    dis = jnp.where(pos, lax.rsqrt(jnp.where(pos, deg, 1.0)), 0.0)
    xw = jnp.dot(x_ref[...], w_ref[...], preferred_element_type=jnp.float32)
    y_ref[...] = xw * dis
    dis_ref[...] = dis


def _dense_call(x, w, degp3):
    return pl.pallas_call(
        _dense_body,
        grid=(N // BN,),
        in_specs=[
            pl.BlockSpec((BN, D), lambda g: (g, 0)),
            pl.BlockSpec((D, D), lambda g: (0, 0)),
            pl.BlockSpec((NC, BN, 1), lambda g: (0, g, 0)),
        ],
        out_specs=[
            pl.BlockSpec((BN, D), lambda g: (g, 0)),
            pl.BlockSpec((BN, 1), lambda g: (g, 0)),
        ],
        out_shape=[
            jax.ShapeDtypeStruct((N, D), jnp.float32),
            jax.ShapeDtypeStruct((N, 1), jnp.float32),
        ],
    )(x, w, degp3)


def _final_body(outp_ref, dis_ref, b_ref, o_ref):
    t = outp_ref[0] + outp_ref[1]
    o_ref[...] = jnp.maximum(t * dis_ref[...] + b_ref[...], 0.0)


def _final_call(outp, dis, b2):
    return pl.pallas_call(
        _final_body,
        grid=(N // BN,),
        in_specs=[
            pl.BlockSpec((NC, BN, D), lambda g: (0, g, 0)),
            pl.BlockSpec((BN, 1), lambda g: (g, 0)),
            pl.BlockSpec((1, D), lambda g: (0, 0)),
        ],
        out_specs=pl.BlockSpec((BN, D), lambda g: (g, 0)),
        out_shape=jax.ShapeDtypeStruct((N, D), jnp.float32),
    )(outp, dis, b2)


def kernel(x, edge_index, edge_weights, W, b):
    nblk = E // (KW * SCROWS)
    ei = edge_index.astype(jnp.int32)
    src3 = ei[0].reshape(nblk, SCROWS, KW)
    dst3 = ei[1].reshape(nblk, SCROWS, KW)
    ew3 = edge_weights.reshape(nblk, SCROWS, KW)
    degp = _deg_call(dst3, ew3)
    y, dis = _dense_call(x, W, degp.reshape(NC, N, 1))
    outp = _agg_call(y, src3, dst3, edge_weights)
    out = _final_call(outp.reshape(NC, N, D), dis, b.reshape(1, D))
    return (out, edge_index, edge_weights)


# speedup vs baseline: 2.0103x; 1.0316x over previous
import functools

import jax
import jax.numpy as jnp
from jax import lax
from jax.experimental import pallas as pl
from jax.experimental.pallas import tpu as pltpu
from jax.experimental.pallas import tpu_sc as plsc

NC, NS, LANES = 2, 16, 16
NW = NC * NS
KW = 80
SCROWS = 25

N = 10000
E = 320000
D = 128
BN = 1000
N2 = 10240
NPT = N // NS
RPW = (E // KW) // NW


def _deg_body(dst2, ew2, degp, idx_d, ew_v, zb, acc1, sem_sc):
    c = lax.axis_index("c")
    s = lax.axis_index("s")
    wid = c * NS + s
    @pl.when(s < 5)
    def _():
        @pl.loop(0, zb.shape[0] // LANES)
        def _(i):
            zb[pl.ds(i * LANES, LANES)] = jnp.zeros((LANES,), jnp.float32)

        pltpu.sync_copy(zb, acc1.at[pl.ds(s * 2048, 2048)])

    plsc.subcore_barrier()

    @pl.loop(0, RPW // SCROWS)
    def _(ci):
        blk = wid * (RPW // SCROWS) + ci
        pltpu.sync_copy(dst2.at[blk], idx_d)
        pltpu.sync_copy(ew2.at[blk], ew_v)
        descs = [
            pltpu.async_copy(ew_v.at[j], acc1.at[idx_d.at[j]], sem_sc, add=True)
            for j in range(SCROWS)
        ]
        for dsc in descs:
            dsc.wait()

    plsc.subcore_barrier()

    @pl.when(s == 0)
    def _():
        pltpu.sync_copy(acc1, degp.at[c, 0])


_deg_call = functools.partial(
    pl.kernel,
    out_type=jax.ShapeDtypeStruct((NC, 1, N2), jnp.float32),
    mesh=plsc.VectorSubcoreMesh(
        core_axis_name="c", subcore_axis_name="s", num_cores=NC, num_subcores=NS
    ),
    scratch_types=[
        pltpu.VMEM((SCROWS, KW), jnp.int32),
        pltpu.VMEM((SCROWS, KW), jnp.float32),
        pltpu.VMEM((2048,), jnp.float32),
        pltpu.VMEM_SHARED((N2,), jnp.float32),
        pltpu.SemaphoreType.DMA,
    ],
    compiler_params=pltpu.CompilerParams(needs_layout_passes=False),
)(_deg_body)


NGB = 3
NSB = 2


def _agg_body(xw, src1, dst2, ew1, degp, outp, gb0, gb1,
              src_v, idx_d, ew_v, pa, pb, dis_v, acc, dis_sh,
              sem_g0, sem_g1, sem_s0, sem_s1):
    c = lax.axis_index("c")
    s = lax.axis_index("s")
    wid = c * NS + s

    @pl.loop(0, KW)
    def _(i):
        for r in range(D // LANES):
            gb0[i, pl.ds(r * LANES, LANES)] = jnp.zeros((LANES,), jnp.float32)

    for k in range(NPT // KW):
        pltpu.sync_copy(gb0, acc.at[pl.ds(s * NPT + k * KW, KW)])
    pltpu.sync_copy(
        gb0.at[pl.ds(0, NPT % KW)],
        acc.at[pl.ds(s * NPT + (NPT // KW) * KW, NPT % KW)],
    )

    dbase = pl.multiple_of(s * 640, 128)
    pltpu.sync_copy(degp.at[0, 0, pl.ds(dbase, 640)], pa)
    pltpu.sync_copy(degp.at[1, 0, pl.ds(dbase, 640)], pb)

    @pl.loop(0, 640 // LANES)
    def _(v):
        dg = pa[pl.ds(v * LANES, LANES)] + pb[pl.ds(v * LANES, LANES)]
        u = plsc.bitcast(dg, jnp.int32)
        m = jnp.int32(0x5F3759DF) - lax.shift_right_logical(u, 1)
        h = plsc.bitcast(m, jnp.float32)
        h = h * (1.5 - 0.5 * dg * h * h)
        h = h * (1.5 - 0.5 * dg * h * h)
        h = h * (1.5 - 0.5 * dg * h * h)
        pa[pl.ds(v * LANES, LANES)] = jnp.where(dg > 0.0, h, 0.0)

    pltpu.sync_copy(pa, dis_sh.at[s])

    plsc.subcore_barrier()
    dcp = [
        pltpu.async_copy(dis_sh.at[r], dis_v.at[pl.ds(r * 640, 640)], sem_s1)
        for r in range(NS)
    ]
    for d in dcp:
        d.wait()

    gbufs = (gb0, gb1)
    gsems = (sem_g0, sem_g1)
    ssems = (sem_s0, sem_s1)

    @pl.loop(0, RPW // SCROWS)
    def _(ci):
        blk = wid * (RPW // SCROWS) + ci
        st1 = pltpu.async_copy(
            src1.at[pl.ds(blk * (SCROWS * KW), SCROWS * KW)], src_v, sem_s0
        )
        st2 = pltpu.async_copy(dst2.at[blk], idx_d, sem_s0)
        st3 = pltpu.async_copy(
            ew1.at[pl.ds(blk * (SCROWS * KW), SCROWS * KW)], ew_v, sem_s0
        )
        st1.wait()
        st2.wait()
        st3.wait()

        @pl.loop(0, (SCROWS * KW) // LANES)
        def _(v):
            sv = src_v[pl.ds(v * LANES, LANES)]
            dv = plsc.load_gather(dis_v, [sv])
            ew_v[pl.ds(v * LANES, LANES)] = ew_v[pl.ds(v * LANES, LANES)] * dv

        gd = [None] * SCROWS
        sd = [None] * SCROWS
        gd[0] = pltpu.async_copy(
            xw.at[src_v.at[pl.ds(0, KW)]], gbufs[0], gsems[0]
        )
        for j in range(SCROWS):
            b = j & 1
            gd[j].wait()
            if j >= 1:
                sd[j - 1].wait()
            if j + 1 < SCROWS:
                gd[j + 1] = pltpu.async_copy(
                    xw.at[src_v.at[pl.ds((j + 1) * KW, KW)]],
                    gbufs[(j + 1) & 1],
                    gsems[(j + 1) & 1],
                )
            gbuf = gbufs[b]

            @pl.loop(0, KW, unroll=2)
            def _(e):
                ews = plsc.load_gather(
                    ew_v, [jnp.full((LANES,), e, jnp.int32) + (j * KW)]
                )
                for r in range(D // LANES):
                    gbuf[e, pl.ds(r * LANES, LANES)] = (
                        gbuf[e, pl.ds(r * LANES, LANES)] * ews
                    )

            sd[j] = pltpu.async_copy(gbuf, acc.at[idx_d.at[j]], ssems[b], add=True)
        sd[SCROWS - 1].wait()

    plsc.subcore_barrier()
    pltpu.sync_copy(acc.at[pl.ds(s * NPT, NPT)], outp.at[c, s])


_agg_call = functools.partial(
    pl.kernel,
    out_type=jax.ShapeDtypeStruct((NC, NS, NPT, D), jnp.float32),
    mesh=plsc.VectorSubcoreMesh(
        core_axis_name="c", subcore_axis_name="s", num_cores=NC, num_subcores=NS
    ),
    scratch_types=[
        pltpu.VMEM((KW, D), jnp.float32),
        pltpu.VMEM((KW, D), jnp.float32),
        pltpu.VMEM((SCROWS * KW,), jnp.int32),
        pltpu.VMEM((SCROWS, KW), jnp.int32),
        pltpu.VMEM((SCROWS * KW,), jnp.float32),
        pltpu.VMEM((640,), jnp.float32),
        pltpu.VMEM((640,), jnp.float32),
        pltpu.VMEM((N2,), jnp.float32),
        pltpu.VMEM_SHARED((N, D), jnp.float32),
        pltpu.VMEM_SHARED((NS, 640), jnp.float32),
        pltpu.SemaphoreType.DMA,
        pltpu.SemaphoreType.DMA,
        pltpu.SemaphoreType.DMA,
        pltpu.SemaphoreType.DMA,
    ],
    compiler_params=pltpu.CompilerParams(needs_layout_passes=False),
)(_agg_body)


def _dense_body(x_ref, w_ref, y_ref):
    y_ref[...] = jnp.dot(x_ref[...], w_ref[...], preferred_element_type=jnp.float32)


def _dense_call(x, w):
    return pl.pallas_call(
        _dense_body,
        grid=(N // BN,),
        in_specs=[
            pl.BlockSpec((BN, D), lambda g: (g, 0)),
            pl.BlockSpec((D, D), lambda g: (0, 0)),
        ],
        out_specs=pl.BlockSpec((BN, D), lambda g: (g, 0)),
        out_shape=jax.ShapeDtypeStruct((N, D), jnp.float32),
    )(x, w)


def _final_body(outp_ref, degp_ref, b_ref, o_ref):
    dp = degp_ref[...]
    deg = dp[0] + dp[1]
    pos = deg > 0.0
    dis = jnp.where(pos, lax.rsqrt(jnp.where(pos, deg, 1.0)), 0.0)
    t = outp_ref[0] + outp_ref[1]
    o_ref[...] = jnp.maximum(t * dis + b_ref[...], 0.0)


def _final_call(outp, degp3, b2):
    return pl.pallas_call(
        _final_body,
        grid=(N // BN,),
        in_specs=[
            pl.BlockSpec((NC, BN, D), lambda g: (0, g, 0)),
            pl.BlockSpec((NC, BN, 1), lambda g: (0, g, 0)),
            pl.BlockSpec((1, D), lambda g: (0, 0)),
        ],
        out_specs=pl.BlockSpec((BN, D), lambda g: (g, 0)),
        out_shape=jax.ShapeDtypeStruct((N, D), jnp.float32),
    )(outp, degp3, b2)


def kernel(x, edge_index, edge_weights, W, b):
    nblk = E // (KW * SCROWS)
    ei = edge_index.astype(jnp.int32)
    dst3 = ei[1].reshape(nblk, SCROWS, KW)
    ew3 = edge_weights.reshape(nblk, SCROWS, KW)
    degp = _deg_call(dst3, ew3)
    xw = _dense_call(x, W)
    outp = _agg_call(xw, ei[0], dst3, edge_weights, degp)
    out = _final_call(
        outp.reshape(NC, N, D), degp.reshape(NC, N2, 1), b.reshape(1, D)
    )
    return (out, edge_index, edge_weights)


# speedup vs baseline: 2.0373x; 1.0135x over previous
import functools

import jax
import jax.numpy as jnp
from jax import lax
from jax.experimental import pallas as pl
from jax.experimental.pallas import tpu as pltpu
from jax.experimental.pallas import tpu_sc as plsc

NC, NS, LANES = 2, 16, 16
NW = NC * NS
KW = 80
SCROWS = 25

N = 10000
E = 320000
D = 128
BN = 1000
N2 = 10240
NPT = N // NS
RPW = (E // KW) // NW


def _deg_body(dst2, ew2, degp, idx_d, ew_v, zb, acc1, sem_sc):
    c = lax.axis_index("c")
    s = lax.axis_index("s")
    wid = c * NS + s
    @pl.when(s < 5)
    def _():
        @pl.loop(0, zb.shape[0] // LANES)
        def _(i):
            zb[pl.ds(i * LANES, LANES)] = jnp.zeros((LANES,), jnp.float32)

        pltpu.sync_copy(zb, acc1.at[pl.ds(s * 2048, 2048)])

    plsc.subcore_barrier()

    @pl.loop(0, RPW // SCROWS)
    def _(ci):
        blk = wid * (RPW // SCROWS) + ci
        pltpu.sync_copy(dst2.at[blk], idx_d)
        pltpu.sync_copy(ew2.at[blk], ew_v)
        descs = [
            pltpu.async_copy(ew_v.at[j], acc1.at[idx_d.at[j]], sem_sc, add=True)
            for j in range(SCROWS)
        ]
        for dsc in descs:
            dsc.wait()

    plsc.subcore_barrier()

    @pl.when(s == 0)
    def _():
        pltpu.sync_copy(acc1, degp.at[c, 0])


_deg_call = functools.partial(
    pl.kernel,
    out_type=jax.ShapeDtypeStruct((NC, 1, N2), jnp.float32),
    mesh=plsc.VectorSubcoreMesh(
        core_axis_name="c", subcore_axis_name="s", num_cores=NC, num_subcores=NS
    ),
    scratch_types=[
        pltpu.VMEM((SCROWS, KW), jnp.int32),
        pltpu.VMEM((SCROWS, KW), jnp.float32),
        pltpu.VMEM((2048,), jnp.float32),
        pltpu.VMEM_SHARED((N2,), jnp.float32),
        pltpu.SemaphoreType.DMA,
    ],
    compiler_params=pltpu.CompilerParams(needs_layout_passes=False),
)(_deg_body)


NGB = 3
NSB = 2


def _agg_body(xw, src1, dst2, ew1, degp, outp, gb0, gb1,
              src_v, idx_d, ew_v, pa, pb, dis_v, acc, dis_sh,
              sem_g0, sem_g1, sem_s0, sem_s1):
    c = lax.axis_index("c")
    s = lax.axis_index("s")
    wid = c * NS + s

    @pl.loop(0, KW)
    def _(i):
        for r in range(D // LANES):
            gb0[i, pl.ds(r * LANES, LANES)] = jnp.zeros((LANES,), jnp.float32)

    for k in range(NPT // KW):
        pltpu.sync_copy(gb0, acc.at[pl.ds(s * NPT + k * KW, KW)])
    pltpu.sync_copy(
        gb0.at[pl.ds(0, NPT % KW)],
        acc.at[pl.ds(s * NPT + (NPT // KW) * KW, NPT % KW)],
    )

    dbase = pl.multiple_of(s * 640, 128)
    pltpu.sync_copy(degp.at[0, 0, pl.ds(dbase, 640)], pa)
    pltpu.sync_copy(degp.at[1, 0, pl.ds(dbase, 640)], pb)

    @pl.loop(0, 640 // LANES)
    def _(v):
        dg = pa[pl.ds(v * LANES, LANES)] + pb[pl.ds(v * LANES, LANES)]
        u = plsc.bitcast(dg, jnp.int32)
        m = jnp.int32(0x5F3759DF) - lax.shift_right_logical(u, 1)
        h = plsc.bitcast(m, jnp.float32)
        h = h * (1.5 - 0.5 * dg * h * h)
        h = h * (1.5 - 0.5 * dg * h * h)
        h = h * (1.5 - 0.5 * dg * h * h)
        pa[pl.ds(v * LANES, LANES)] = jnp.where(dg > 0.0, h, 0.0)

    pltpu.sync_copy(pa, dis_sh.at[s])

    plsc.subcore_barrier()
    dcp = [
        pltpu.async_copy(dis_sh.at[r], dis_v.at[pl.ds(r * 640, 640)], sem_s1)
        for r in range(NS)
    ]
    for d in dcp:
        d.wait()

    gbufs = (gb0, gb1)
    gsems = (sem_g0, sem_g1)
    ssems = (sem_s0, sem_s1)

    @pl.loop(0, RPW // SCROWS)
    def _(ci):
        blk = wid * (RPW // SCROWS) + ci
        st1 = pltpu.async_copy(
            src1.at[pl.ds(blk * (SCROWS * KW), SCROWS * KW)], src_v, sem_s0
        )
        st2 = pltpu.async_copy(dst2.at[blk], idx_d, sem_s0)
        st3 = pltpu.async_copy(
            ew1.at[pl.ds(blk * (SCROWS * KW), SCROWS * KW)], ew_v, sem_s0
        )
        st1.wait()
        gd = [None] * SCROWS
        sd = [None] * SCROWS
        gd[0] = pltpu.async_copy(
            xw.at[src_v.at[pl.ds(0, KW)]], gbufs[0], gsems[0]
        )
        gd[1] = pltpu.async_copy(
            xw.at[src_v.at[pl.ds(KW, KW)]], gbufs[1], gsems[1]
        )
        st3.wait()

        @pl.loop(0, (SCROWS * KW) // LANES)
        def _(v):
            sv = src_v[pl.ds(v * LANES, LANES)]
            dv = plsc.load_gather(dis_v, [sv])
            ew_v[pl.ds(v * LANES, LANES)] = ew_v[pl.ds(v * LANES, LANES)] * dv

        st2.wait()
        for j in range(SCROWS):
            b = j & 1
            gd[j].wait()
            if j >= 1:
                sd[j - 1].wait()
            if j >= 1 and j + 1 < SCROWS:
                gd[j + 1] = pltpu.async_copy(
                    xw.at[src_v.at[pl.ds((j + 1) * KW, KW)]],
                    gbufs[(j + 1) & 1],
                    gsems[(j + 1) & 1],
                )
            gbuf = gbufs[b]

            @pl.loop(0, KW, unroll=2)
            def _(e):
                ews = plsc.load_gather(
                    ew_v, [jnp.full((LANES,), e, jnp.int32) + (j * KW)]
                )
                for r in range(D // LANES):
                    gbuf[e, pl.ds(r * LANES, LANES)] = (
                        gbuf[e, pl.ds(r * LANES, LANES)] * ews
                    )

            sd[j] = pltpu.async_copy(gbuf, acc.at[idx_d.at[j]], ssems[b], add=True)
        sd[SCROWS - 1].wait()

    plsc.subcore_barrier()
    pltpu.sync_copy(acc.at[pl.ds(s * NPT, NPT)], outp.at[c, s])


_agg_call = functools.partial(
    pl.kernel,
    out_type=jax.ShapeDtypeStruct((NC, NS, NPT, D), jnp.float32),
    mesh=plsc.VectorSubcoreMesh(
        core_axis_name="c", subcore_axis_name="s", num_cores=NC, num_subcores=NS
    ),
    scratch_types=[
        pltpu.VMEM((KW, D), jnp.float32),
        pltpu.VMEM((KW, D), jnp.float32),
        pltpu.VMEM((SCROWS * KW,), jnp.int32),
        pltpu.VMEM((SCROWS, KW), jnp.int32),
        pltpu.VMEM((SCROWS * KW,), jnp.float32),
        pltpu.VMEM((640,), jnp.float32),
        pltpu.VMEM((640,), jnp.float32),
        pltpu.VMEM((N2,), jnp.float32),
        pltpu.VMEM_SHARED((N, D), jnp.float32),
        pltpu.VMEM_SHARED((NS, 640), jnp.float32),
        pltpu.SemaphoreType.DMA,
        pltpu.SemaphoreType.DMA,
        pltpu.SemaphoreType.DMA,
        pltpu.SemaphoreType.DMA,
    ],
    compiler_params=pltpu.CompilerParams(needs_layout_passes=False),
)(_agg_body)


def _dense_body(x_ref, w_ref, y_ref):
    y_ref[...] = jnp.dot(x_ref[...], w_ref[...], preferred_element_type=jnp.float32)


def _dense_call(x, w):
    return pl.pallas_call(
        _dense_body,
        grid=(N // BN,),
        in_specs=[
            pl.BlockSpec((BN, D), lambda g: (g, 0)),
            pl.BlockSpec((D, D), lambda g: (0, 0)),
        ],
        out_specs=pl.BlockSpec((BN, D), lambda g: (g, 0)),
        out_shape=jax.ShapeDtypeStruct((N, D), jnp.float32),
    )(x, w)


def _final_body(outp_ref, degp_ref, b_ref, o_ref):
    dp = degp_ref[...]
    deg = dp[0] + dp[1]
    pos = deg > 0.0
    dis = jnp.where(pos, lax.rsqrt(jnp.where(pos, deg, 1.0)), 0.0)
    t = outp_ref[0] + outp_ref[1]
    o_ref[...] = jnp.maximum(t * dis + b_ref[...], 0.0)


def _final_call(outp, degp3, b2):
    return pl.pallas_call(
        _final_body,
        grid=(N // BN,),
        in_specs=[
            pl.BlockSpec((NC, BN, D), lambda g: (0, g, 0)),
            pl.BlockSpec((NC, BN, 1), lambda g: (0, g, 0)),
            pl.BlockSpec((1, D), lambda g: (0, 0)),
        ],
        out_specs=pl.BlockSpec((BN, D), lambda g: (g, 0)),
        out_shape=jax.ShapeDtypeStruct((N, D), jnp.float32),
    )(outp, degp3, b2)


def kernel(x, edge_index, edge_weights, W, b):
    nblk = E // (KW * SCROWS)
    ei = edge_index.astype(jnp.int32)
    dst3 = ei[1].reshape(nblk, SCROWS, KW)
    ew3 = edge_weights.reshape(nblk, SCROWS, KW)
    degp = _deg_call(dst3, ew3)
    xw = _dense_call(x, W)
    outp = _agg_call(xw, ei[0], dst3, edge_weights, degp)
    out = _final_call(
        outp.reshape(NC, N, D), degp.reshape(NC, N2, 1), b.reshape(1, D)
    )
    return (out, edge_index, edge_weights)
